# baseline (speedup 1.0000x reference)
import functools

import jax
import jax.numpy as jnp
from jax import lax
from jax.experimental import pallas as pl
from jax.experimental.pallas import tpu as pltpu
from jax.experimental.pallas import tpu_sc as plsc

N = 10000
E = 320000
D = 128
DH = D // 2
NC, NS, L = 2, 16, 16
NW = NC * NS
CH = 128
CHS = 64
NP = 10240
RPT = NP // NS
EPAD = 327680
NCHD = 80
NCHS = 320
NB = 16
CW = 8
BR = 400
GR = N // BR

_mesh = plsc.VectorSubcoreMesh(
    core_axis_name="c", subcore_axis_name="s", num_cores=NC, num_subcores=NS)
_sc_params = pltpu.CompilerParams(use_tc_tiling_on_sc=False)


NBD = 4


@functools.partial(
    pl.kernel,
    out_type=(jax.ShapeDtypeStruct((NC, NP, CW), jnp.float32),
              jax.ShapeDtypeStruct((NC, NP, CW), jnp.float32)),
    mesh=_mesh,
    compiler_params=_sc_params,
    scratch_types=[
        pltpu.VMEM((NBD, CH), jnp.int32),
        pltpu.VMEM((CH, CW), jnp.float32),
        pltpu.VMEM_SHARED((NP, CW), jnp.float32),
    ],
)
def _deg_kernel(src_hbm, dst_hbm, out_src, out_dst, idx_v, fill_v, acc):
    c = lax.axis_index("c")
    s = lax.axis_index("s")
    w = s * NC + c

    def _fill(val):
        @pl.loop(0, CH)
        def _(r):
            fill_v[r, :] = jnp.full((CW,), val, jnp.float32)

    def _zero_stripe():
        _fill(0.0)

        @pl.loop(0, RPT // CH)
        def _(k):
            pltpu.sync_copy(fill_v, acc.at[pl.ds(s * RPT + k * CH, CH)])
        plsc.subcore_barrier()
        _fill(1.0)

    def _count(idx_hbm, out_hbm):
        @pl.loop(0, NCHD // NBD)
        def _(b):
            pltpu.sync_copy(idx_hbm.at[w, pl.ds(b * NBD, NBD)], idx_v)

            @pl.loop(0, NBD)
            def _(j):
                pltpu.sync_copy(fill_v, acc.at[idx_v.at[j]], add=True)

        plsc.subcore_barrier()
        pltpu.sync_copy(acc.at[pl.ds(s * RPT, RPT)],
                        out_hbm.at[c, pl.ds(s * RPT, RPT)])

    _zero_stripe()
    _count(src_hbm, out_src)
    _zero_stripe()
    _count(dst_hbm, out_dst)


NBLK = NCHS // NB
DEPTH = 4


@functools.partial(
    pl.kernel,
    out_type=jax.ShapeDtypeStruct((NC, NP, DH), jnp.float32),
    mesh=_mesh,
    compiler_params=_sc_params,
    scratch_types=[
        pltpu.VMEM((NB, CHS), jnp.int32),
        pltpu.VMEM((NB, CHS), jnp.int32),
        pltpu.VMEM((NB, CHS), jnp.int32),
        pltpu.VMEM((NB, CHS), jnp.int32),
        pltpu.VMEM((CHS, DH), jnp.float32),
        pltpu.VMEM((CHS, DH), jnp.float32),
        pltpu.VMEM((CHS, DH), jnp.float32),
        pltpu.VMEM((CHS, DH), jnp.float32),
        pltpu.SemaphoreType.DMA,
        pltpu.SemaphoreType.DMA,
        pltpu.SemaphoreType.DMA,
        pltpu.SemaphoreType.DMA,
        pltpu.SemaphoreType.DMA,
        pltpu.SemaphoreType.DMA,
        pltpu.SemaphoreType.DMA,
        pltpu.SemaphoreType.DMA,
        pltpu.SemaphoreType.DMA,
        pltpu.VMEM_SHARED((NP, DH), jnp.float32),
    ],
)
def _spmm_kernel(table_hbm, srcoff_hbm, dst_hbm, out_hbm,
                 srcA, dstA, srcB, dstB, rows0, rows1, rows2, rows3,
                 sg0, sg1, sg2, sg3, ss0, ss1, ss2, ss3, si, acc):
    c = lax.axis_index("c")
    s = lax.axis_index("s")
    rows = (rows0, rows1, rows2, rows3)
    sg = (sg0, sg1, sg2, sg3)
    ss = (ss0, ss1, ss2, ss3)

    @pl.loop(0, CHS)
    def _(r):
        for g in range(DH // L):
            rows0[r, pl.ds(g * L, L)] = jnp.zeros((L,), jnp.float32)
            rows1[r, pl.ds(g * L, L)] = jnp.zeros((L,), jnp.float32)

    @pl.loop(0, RPT // (2 * CHS))
    def _(k):
        pltpu.sync_copy(rows0, acc.at[pl.ds(s * RPT + k * 2 * CHS, CHS)])
        pltpu.sync_copy(rows1, acc.at[pl.ds(s * RPT + k * 2 * CHS + CHS, CHS)])
    plsc.subcore_barrier()

    def stage_idx_async(bb, sv, dv):
        pltpu.async_copy(srcoff_hbm.at[c, s, pl.ds(bb * NB, NB)], sv, si)
        pltpu.async_copy(dst_hbm.at[s, pl.ds(bb * NB, NB)], dv, si)

    def wait_idx(sv, dv):
        pltpu.make_async_copy(srcoff_hbm.at[c, s, pl.ds(0, NB)], sv, si).wait()
        pltpu.make_async_copy(dst_hbm.at[s, pl.ds(0, NB)], dv, si).wait()

    def gather_issue(idx_row, p):
        pltpu.async_copy(table_hbm.at[idx_row], rows[p], sg[p])

    def gather_wait(sv, p):
        pltpu.make_async_copy(table_hbm.at[pl.ds(0, CHS)], rows[p], sg[p]).wait()

    def scatter_issue(dx, jj, p):
        pltpu.async_copy(rows[p], acc.at[dx.at[jj]], ss[p], add=True)

    def scatter_wait(dx, p):
        pltpu.make_async_copy(rows[p], acc.at[pl.ds(0, CHS)], ss[p]).wait()

    LA = DEPTH - 1

    def do_block(bb, sx, dx, sy, dy, prefetch, last, first=False):
        if prefetch:
            stage_idx_async(bb + 1, sy, dy)
        for jj in range(NB):
            p = jj % DEPTH
            if jj == NB - LA and prefetch:
                wait_idx(sy, dy)
            gather_wait(sx, p)
            if not (last and jj >= NB - LA):
                q = (jj + LA) % DEPTH
                if not (first and jj == 0):
                    scatter_wait(dx, q)
                nidx = sx.at[jj + LA] if jj < NB - LA else sy.at[jj - (NB - LA)]
                gather_issue(nidx, q)
            scatter_issue(dx, jj, p)
        if last:
            for p in range(DEPTH):
                scatter_wait(dx, p)

    pltpu.sync_copy(srcoff_hbm.at[c, s, pl.ds(0, NB)], srcA)
    pltpu.sync_copy(dst_hbm.at[s, pl.ds(0, NB)], dstA)
    for j0 in range(LA):
        gather_issue(srcA.at[j0], j0)

    do_block(0, srcA, dstA, srcB, dstB, prefetch=True, last=False, first=True)

    @pl.loop(0, NBLK // 2 - 1)
    def _(m):
        do_block(2 * m + 1, srcB, dstB, srcA, dstA, prefetch=True, last=False)
        do_block(2 * m + 2, srcA, dstA, srcB, dstB, prefetch=True, last=False)

    do_block(NBLK - 1, srcB, dstB, srcA, dstA, prefetch=False, last=True)

    plsc.subcore_barrier()
    pltpu.sync_copy(acc.at[pl.ds(s * RPT, RPT)],
                    out_hbm.at[c, pl.ds(s * RPT, RPT)])


def _wvec(degs):
    return lax.rsqrt(jnp.maximum(degs[0, :, 0] + degs[1, :, 0], 1.0))


def _split(h):
    return jnp.stack([h[:, :DH], h[:, DH:]], axis=0)


def _linear_scale_body(x_ref, w_ref, b_ref, degs_ref, out_ref):
    h = jnp.dot(x_ref[...], w_ref[...],
                preferred_element_type=jnp.float32) + b_ref[...]
    wo = _wvec(degs_ref[...])
    out_ref[...] = _split(h * wo[:, None])


def _linear_scale(x, W, b, degp_src):
    return pl.pallas_call(
        _linear_scale_body,
        grid=(GR,),
        in_specs=[
            pl.BlockSpec((BR, D), lambda i: (i, 0)),
            pl.BlockSpec((D, D), lambda i: (0, 0)),
            pl.BlockSpec((1, D), lambda i: (0, 0)),
            pl.BlockSpec((NC, BR, CW), lambda i: (0, i, 0)),
        ],
        out_specs=pl.BlockSpec((NC, BR, DH), lambda i: (0, i, 0)),
        out_shape=jax.ShapeDtypeStruct((NC, N, DH), jnp.float32),
    )(x, W, b.reshape(1, D), degp_src)


def _combine_linear_body(p_ref, degd_ref, degs_ref, w_ref, b_ref, out_ref):
    p = p_ref[...]
    sup = jnp.concatenate([p[0], p[1]], axis=-1)
    wi = _wvec(degd_ref[...])
    h = jnp.maximum(sup * wi[:, None], 0.0)
    wo = _wvec(degs_ref[...])
    out_ref[...] = _split(
        (jnp.dot(h, w_ref[...], preferred_element_type=jnp.float32)
         + b_ref[...]) * wo[:, None])


def _combine_linear(p, degp_dst, degp_src, W, b):
    return pl.pallas_call(
        _combine_linear_body,
        grid=(GR,),
        in_specs=[
            pl.BlockSpec((NC, BR, DH), lambda i: (0, i, 0)),
            pl.BlockSpec((NC, BR, CW), lambda i: (0, i, 0)),
            pl.BlockSpec((NC, BR, CW), lambda i: (0, i, 0)),
            pl.BlockSpec((D, D), lambda i: (0, 0)),
            pl.BlockSpec((1, D), lambda i: (0, 0)),
        ],
        out_specs=pl.BlockSpec((NC, BR, DH), lambda i: (0, i, 0)),
        out_shape=jax.ShapeDtypeStruct((NC, N, DH), jnp.float32),
    )(p, degp_dst, degp_src, W, b.reshape(1, D))


def _combine_relu_body(p_ref, degd_ref, out_ref):
    p = p_ref[...]
    sup = jnp.concatenate([p[0], p[1]], axis=-1)
    wi = _wvec(degd_ref[...])
    out_ref[...] = jnp.maximum(sup * wi[:, None], 0.0)


def _combine_relu(p, degp_dst):
    return pl.pallas_call(
        _combine_relu_body,
        grid=(GR,),
        in_specs=[
            pl.BlockSpec((NC, BR, DH), lambda i: (0, i, 0)),
            pl.BlockSpec((NC, BR, CW), lambda i: (0, i, 0)),
        ],
        out_specs=pl.BlockSpec((BR, D), lambda i: (i, 0)),
        out_shape=jax.ShapeDtypeStruct((N, D), jnp.float32),
    )(p, degp_dst)


def kernel(x, edge_index, W1, b1, W2, b2):
    src = edge_index[0]
    dst = edge_index[1]
    fill = jnp.full((EPAD - E,), NP - 1, jnp.int32)
    dstp = jnp.concatenate([dst, fill])
    src_d = jnp.concatenate([src, fill]).reshape(NW, NCHD, CH)
    dst_d = dstp.reshape(NW, NCHD, CH)
    srcp = jnp.concatenate([src, jnp.zeros((EPAD - E,), jnp.int32)])
    srcoff = (srcp[None, :] + (jnp.arange(NC, dtype=jnp.int32) * N)[:, None]
              ).reshape(NC, NS, NCHS, CHS)
    dst_s = dstp.reshape(NS, NCHS, CHS)

    degp_src, degp_dst = _deg_kernel(src_d, dst_d)
    h1s = _linear_scale(x, W1, b1, degp_src)
    p1 = _spmm_kernel(h1s.reshape(NC * N, DH), srcoff, dst_s)
    h2s = _combine_linear(p1, degp_dst, degp_src, W2, b2)
    p2 = _spmm_kernel(h2s.reshape(NC * N, DH), srcoff, dst_s)
    return _combine_relu(p2, degp_dst)


# speedup vs baseline: 1.0671x; 1.0671x over previous
import functools

import jax
import jax.numpy as jnp
from jax import lax
from jax.experimental import pallas as pl
from jax.experimental.pallas import tpu as pltpu
from jax.experimental.pallas import tpu_sc as plsc

N = 10000
E = 320000
D = 128
DH = D // 2
NC, NS, L = 2, 16, 16
NW = NC * NS
CH = 128
CHS = 64
NP = 10240
RPT = NP // NS
EPAD = 327680
NCHD = 80
NCHS = 320
NB = 16
CW = 8
BR = 512
GR = NP // BR

_mesh = plsc.VectorSubcoreMesh(
    core_axis_name="c", subcore_axis_name="s", num_cores=NC, num_subcores=NS)
_sc_params = pltpu.CompilerParams(use_tc_tiling_on_sc=False)


NBD = 4


@functools.partial(
    pl.kernel,
    out_type=(jax.ShapeDtypeStruct((NC, NP, CW), jnp.float32),
              jax.ShapeDtypeStruct((NC, NP, CW), jnp.float32)),
    mesh=_mesh,
    compiler_params=_sc_params,
    scratch_types=[
        pltpu.VMEM((NBD, CH), jnp.int32),
        pltpu.VMEM((CH, CW), jnp.float32),
        pltpu.VMEM_SHARED((NP, CW), jnp.float32),
    ],
)
def _deg_kernel(src_hbm, dst_hbm, out_src, out_dst, idx_v, fill_v, acc):
    c = lax.axis_index("c")
    s = lax.axis_index("s")
    w = s * NC + c

    def _fill(val):
        @pl.loop(0, CH)
        def _(r):
            fill_v[r, :] = jnp.full((CW,), val, jnp.float32)

    def _zero_stripe():
        _fill(0.0)

        @pl.loop(0, RPT // CH)
        def _(k):
            pltpu.sync_copy(fill_v, acc.at[pl.ds(s * RPT + k * CH, CH)])
        plsc.subcore_barrier()
        _fill(1.0)

    def _count(idx_hbm, out_hbm):
        @pl.loop(0, NCHD // NBD)
        def _(b):
            pltpu.sync_copy(idx_hbm.at[w, pl.ds(b * NBD, NBD)], idx_v)

            @pl.loop(0, NBD)
            def _(j):
                pltpu.sync_copy(fill_v, acc.at[idx_v.at[j]], add=True)

        plsc.subcore_barrier()
        pltpu.sync_copy(acc.at[pl.ds(s * RPT, RPT)],
                        out_hbm.at[c, pl.ds(s * RPT, RPT)])

    _zero_stripe()
    _count(src_hbm, out_src)
    _zero_stripe()
    _count(dst_hbm, out_dst)


NBLK = NCHS // NB
DEPTH = 4


@functools.partial(
    pl.kernel,
    out_type=jax.ShapeDtypeStruct((NC, NP, DH), jnp.float32),
    mesh=_mesh,
    compiler_params=_sc_params,
    scratch_types=[
        pltpu.VMEM((NB, CHS), jnp.int32),
        pltpu.VMEM((NB, CHS), jnp.int32),
        pltpu.VMEM((NB, CHS), jnp.int32),
        pltpu.VMEM((NB, CHS), jnp.int32),
        pltpu.VMEM((CHS, DH), jnp.float32),
        pltpu.VMEM((CHS, DH), jnp.float32),
        pltpu.VMEM((CHS, DH), jnp.float32),
        pltpu.VMEM((CHS, DH), jnp.float32),
        pltpu.SemaphoreType.DMA,
        pltpu.SemaphoreType.DMA,
        pltpu.SemaphoreType.DMA,
        pltpu.SemaphoreType.DMA,
        pltpu.SemaphoreType.DMA,
        pltpu.SemaphoreType.DMA,
        pltpu.SemaphoreType.DMA,
        pltpu.SemaphoreType.DMA,
        pltpu.SemaphoreType.DMA,
        pltpu.VMEM_SHARED((NP, DH), jnp.float32),
    ],
)
def _spmm_kernel(table_hbm, srcoff_hbm, dst_hbm, out_hbm,
                 srcA, dstA, srcB, dstB, rows0, rows1, rows2, rows3,
                 sg0, sg1, sg2, sg3, ss0, ss1, ss2, ss3, si, acc):
    c = lax.axis_index("c")
    s = lax.axis_index("s")
    rows = (rows0, rows1, rows2, rows3)
    sg = (sg0, sg1, sg2, sg3)
    ss = (ss0, ss1, ss2, ss3)

    @pl.loop(0, CHS)
    def _(r):
        for g in range(DH // L):
            rows0[r, pl.ds(g * L, L)] = jnp.zeros((L,), jnp.float32)
            rows1[r, pl.ds(g * L, L)] = jnp.zeros((L,), jnp.float32)

    @pl.loop(0, RPT // (2 * CHS))
    def _(k):
        pltpu.sync_copy(rows0, acc.at[pl.ds(s * RPT + k * 2 * CHS, CHS)])
        pltpu.sync_copy(rows1, acc.at[pl.ds(s * RPT + k * 2 * CHS + CHS, CHS)])
    plsc.subcore_barrier()

    def stage_idx_async(bb, sv, dv):
        pltpu.async_copy(srcoff_hbm.at[c, s, pl.ds(bb * NB, NB)], sv, si)
        pltpu.async_copy(dst_hbm.at[s, pl.ds(bb * NB, NB)], dv, si)

    def wait_idx(sv, dv):
        pltpu.make_async_copy(srcoff_hbm.at[c, s, pl.ds(0, NB)], sv, si).wait()
        pltpu.make_async_copy(dst_hbm.at[s, pl.ds(0, NB)], dv, si).wait()

    def gather_issue(idx_row, p):
        pltpu.async_copy(table_hbm.at[idx_row], rows[p], sg[p])

    def gather_wait(sv, p):
        pltpu.make_async_copy(table_hbm.at[pl.ds(0, CHS)], rows[p], sg[p]).wait()

    def scatter_issue(dx, jj, p):
        pltpu.async_copy(rows[p], acc.at[dx.at[jj]], ss[p], add=True)

    def scatter_wait(dx, p):
        pltpu.make_async_copy(rows[p], acc.at[pl.ds(0, CHS)], ss[p]).wait()

    LA = DEPTH - 1

    def do_block(bb, sx, dx, sy, dy, prefetch, last, first=False):
        if prefetch:
            stage_idx_async(bb + 1, sy, dy)
        for jj in range(NB):
            p = jj % DEPTH
            if jj == NB - LA and prefetch:
                wait_idx(sy, dy)
            gather_wait(sx, p)
            if not (last and jj >= NB - LA):
                q = (jj + LA) % DEPTH
                if not (first and jj == 0):
                    scatter_wait(dx, q)
                nidx = sx.at[jj + LA] if jj < NB - LA else sy.at[jj - (NB - LA)]
                gather_issue(nidx, q)
            scatter_issue(dx, jj, p)
        if last:
            for p in range(DEPTH):
                scatter_wait(dx, p)

    pltpu.sync_copy(srcoff_hbm.at[c, s, pl.ds(0, NB)], srcA)
    pltpu.sync_copy(dst_hbm.at[s, pl.ds(0, NB)], dstA)
    for j0 in range(LA):
        gather_issue(srcA.at[j0], j0)

    do_block(0, srcA, dstA, srcB, dstB, prefetch=True, last=False, first=True)

    @pl.loop(0, NBLK // 2 - 1)
    def _(m):
        do_block(2 * m + 1, srcB, dstB, srcA, dstA, prefetch=True, last=False)
        do_block(2 * m + 2, srcA, dstA, srcB, dstB, prefetch=True, last=False)

    do_block(NBLK - 1, srcB, dstB, srcA, dstA, prefetch=False, last=True)

    plsc.subcore_barrier()
    pltpu.sync_copy(acc.at[pl.ds(s * RPT, RPT)],
                    out_hbm.at[c, pl.ds(s * RPT, RPT)])


def _wvec(degs):
    return lax.rsqrt(jnp.maximum(degs[0, :, 0] + degs[1, :, 0], 1.0))


def _split(h):
    return jnp.stack([h[:, :DH], h[:, DH:]], axis=0)


def _linear_scale_body(x_ref, w_ref, b_ref, degs_ref, out_ref):
    h = jnp.dot(x_ref[...], w_ref[...],
                preferred_element_type=jnp.float32) + b_ref[...]
    wo = _wvec(degs_ref[...])
    out_ref[...] = _split(h * wo[:, None])


def _linear_scale(x, W, b, degp_src):
    return pl.pallas_call(
        _linear_scale_body,
        grid=(GR,),
        in_specs=[
            pl.BlockSpec((BR, D), lambda i: (i, 0)),
            pl.BlockSpec((D, D), lambda i: (0, 0)),
            pl.BlockSpec((1, D), lambda i: (0, 0)),
            pl.BlockSpec((NC, BR, CW), lambda i: (0, i, 0)),
        ],
        out_specs=pl.BlockSpec((NC, BR, DH), lambda i: (0, i, 0)),
        out_shape=jax.ShapeDtypeStruct((NC, NP, DH), jnp.float32),
    )(x, W, b.reshape(1, D), degp_src)


def _combine_linear_body(p_ref, degd_ref, degs_ref, w_ref, b_ref, out_ref):
    p = p_ref[...]
    sup = jnp.concatenate([p[0], p[1]], axis=-1)
    wi = _wvec(degd_ref[...])
    h = jnp.maximum(sup * wi[:, None], 0.0)
    wo = _wvec(degs_ref[...])
    out_ref[...] = _split(
        (jnp.dot(h, w_ref[...], preferred_element_type=jnp.float32)
         + b_ref[...]) * wo[:, None])


def _combine_linear(p, degp_dst, degp_src, W, b):
    return pl.pallas_call(
        _combine_linear_body,
        grid=(GR,),
        in_specs=[
            pl.BlockSpec((NC, BR, DH), lambda i: (0, i, 0)),
            pl.BlockSpec((NC, BR, CW), lambda i: (0, i, 0)),
            pl.BlockSpec((NC, BR, CW), lambda i: (0, i, 0)),
            pl.BlockSpec((D, D), lambda i: (0, 0)),
            pl.BlockSpec((1, D), lambda i: (0, 0)),
        ],
        out_specs=pl.BlockSpec((NC, BR, DH), lambda i: (0, i, 0)),
        out_shape=jax.ShapeDtypeStruct((NC, NP, DH), jnp.float32),
    )(p, degp_dst, degp_src, W, b.reshape(1, D))


def _combine_relu_body(p_ref, degd_ref, out_ref):
    p = p_ref[...]
    sup = jnp.concatenate([p[0], p[1]], axis=-1)
    wi = _wvec(degd_ref[...])
    out_ref[...] = jnp.maximum(sup * wi[:, None], 0.0)


def _combine_relu(p, degp_dst):
    return pl.pallas_call(
        _combine_relu_body,
        grid=(GR,),
        in_specs=[
            pl.BlockSpec((NC, BR, DH), lambda i: (0, i, 0)),
            pl.BlockSpec((NC, BR, CW), lambda i: (0, i, 0)),
        ],
        out_specs=pl.BlockSpec((BR, D), lambda i: (i, 0)),
        out_shape=jax.ShapeDtypeStruct((NP, D), jnp.float32),
    )(p, degp_dst)


def kernel(x, edge_index, W1, b1, W2, b2):
    src = edge_index[0]
    dst = edge_index[1]
    fill = jnp.full((EPAD - E,), NP - 1, jnp.int32)
    srcp = jnp.concatenate([src, fill])
    dstp = jnp.concatenate([dst, fill])
    src_d = srcp.reshape(NW, NCHD, CH)
    dst_d = dstp.reshape(NW, NCHD, CH)
    srcoff = (srcp[None, :] + (jnp.arange(NC, dtype=jnp.int32) * NP)[:, None]
              ).reshape(NC, NS, NCHS, CHS)
    dst_s = dstp.reshape(NS, NCHS, CHS)
    xp = jnp.pad(x, ((0, NP - N), (0, 0)))

    degp_src, degp_dst = _deg_kernel(src_d, dst_d)
    h1s = _linear_scale(xp, W1, b1, degp_src)
    p1 = _spmm_kernel(h1s.reshape(NC * NP, DH), srcoff, dst_s)
    h2s = _combine_linear(p1, degp_dst, degp_src, W2, b2)
    p2 = _spmm_kernel(h2s.reshape(NC * NP, DH), srcoff, dst_s)
    out = _combine_relu(p2, degp_dst)
    return out[:N]


# speedup vs baseline: 1.0887x; 1.0203x over previous
import functools

import jax
import jax.numpy as jnp
from jax import lax
from jax.experimental import pallas as pl
from jax.experimental.pallas import tpu as pltpu
from jax.experimental.pallas import tpu_sc as plsc

N = 10000
E = 320000
D = 128
DH = D // 2
NC, NS, L = 2, 16, 16
NW = NC * NS
CH = 128
CHS = 64
NP = 10240
RPT = NP // NS
EPAD = 327680
NCHD = 80
NCHS = 320
NB = 16
CW = 8
BR = 512
GR = NP // BR

_mesh = plsc.VectorSubcoreMesh(
    core_axis_name="c", subcore_axis_name="s", num_cores=NC, num_subcores=NS)
_sc_params = pltpu.CompilerParams(use_tc_tiling_on_sc=False)


NBD = 4


@functools.partial(
    pl.kernel,
    out_type=(jax.ShapeDtypeStruct((NC, NP, CW), jnp.float32),
              jax.ShapeDtypeStruct((NC, NP, CW), jnp.float32)),
    mesh=_mesh,
    compiler_params=_sc_params,
    scratch_types=[
        pltpu.VMEM((NBD, CH), jnp.int32),
        pltpu.VMEM((NBD, CH), jnp.int32),
        pltpu.VMEM((CH, CW), jnp.float32),
        pltpu.SemaphoreType.DMA,
        pltpu.SemaphoreType.DMA,
        pltpu.VMEM_SHARED((NP, CW), jnp.float32),
    ],
)
def _deg_kernel(src_hbm, dst_hbm, out_src, out_dst,
                idxA, idxB, fill_v, sc_sem, si_sem, acc):
    c = lax.axis_index("c")
    s = lax.axis_index("s")
    w = s * NC + c

    def _fill(val):
        @pl.loop(0, CH)
        def _(r):
            fill_v[r, :] = jnp.full((CW,), val, jnp.float32)

    def _zero_stripe():
        _fill(0.0)

        @pl.loop(0, RPT // CH)
        def _(k):
            pltpu.sync_copy(fill_v, acc.at[pl.ds(s * RPT + k * CH, CH)])
        plsc.subcore_barrier()
        _fill(1.0)

    def _stage(idx_hbm, bb, buf):
        pltpu.async_copy(idx_hbm.at[w, pl.ds(bb * NBD, NBD)], buf, si_sem)

    def _stage_wait(idx_hbm, buf):
        pltpu.make_async_copy(idx_hbm.at[w, pl.ds(0, NBD)], buf, si_sem).wait()

    def _scat_drain():
        for _ in range(NBD):
            pltpu.make_async_copy(fill_v.at[pl.ds(0, CH)],
                                  acc.at[pl.ds(0, CH)], sc_sem).wait()

    def _block(idx_hbm, bb, cur, nxt, prefetch, first):
        if prefetch:
            _stage(idx_hbm, bb + 1, nxt)
        if not first:
            _scat_drain()
        for j in range(NBD):
            pltpu.async_copy(fill_v, acc.at[cur.at[j]], sc_sem, add=True)

    def _count(idx_hbm, out_hbm):
        pltpu.sync_copy(idx_hbm.at[w, pl.ds(0, NBD)], idxA)
        _block(idx_hbm, 0, idxA, idxB, prefetch=True, first=True)

        @pl.loop(0, (NCHD // NBD) // 2 - 1)
        def _(m):
            _stage_wait(idx_hbm, idxB)
            _block(idx_hbm, 2 * m + 1, idxB, idxA, prefetch=True, first=False)
            _stage_wait(idx_hbm, idxA)
            _block(idx_hbm, 2 * m + 2, idxA, idxB, prefetch=True, first=False)

        _stage_wait(idx_hbm, idxB)
        _block(idx_hbm, NCHD // NBD - 1, idxB, idxA, prefetch=False,
               first=False)
        _scat_drain()
        plsc.subcore_barrier()
        pltpu.sync_copy(acc.at[pl.ds(s * RPT, RPT)],
                        out_hbm.at[c, pl.ds(s * RPT, RPT)])

    _zero_stripe()
    _count(src_hbm, out_src)
    _zero_stripe()
    _count(dst_hbm, out_dst)


NBLK = NCHS // NB
DEPTH = 4


@functools.partial(
    pl.kernel,
    out_type=jax.ShapeDtypeStruct((NC, NP, DH), jnp.float32),
    mesh=_mesh,
    compiler_params=_sc_params,
    scratch_types=[
        pltpu.VMEM((NB, CHS), jnp.int32),
        pltpu.VMEM((NB, CHS), jnp.int32),
        pltpu.VMEM((NB, CHS), jnp.int32),
        pltpu.VMEM((NB, CHS), jnp.int32),
        pltpu.VMEM((CHS, DH), jnp.float32),
        pltpu.VMEM((CHS, DH), jnp.float32),
        pltpu.VMEM((CHS, DH), jnp.float32),
        pltpu.VMEM((CHS, DH), jnp.float32),
        pltpu.SemaphoreType.DMA,
        pltpu.SemaphoreType.DMA,
        pltpu.SemaphoreType.DMA,
        pltpu.SemaphoreType.DMA,
        pltpu.SemaphoreType.DMA,
        pltpu.SemaphoreType.DMA,
        pltpu.SemaphoreType.DMA,
        pltpu.SemaphoreType.DMA,
        pltpu.SemaphoreType.DMA,
        pltpu.VMEM_SHARED((NP, DH), jnp.float32),
    ],
)
def _spmm_kernel(table_hbm, srcoff_hbm, dst_hbm, out_hbm,
                 srcA, dstA, srcB, dstB, rows0, rows1, rows2, rows3,
                 sg0, sg1, sg2, sg3, ss0, ss1, ss2, ss3, si, acc):
    c = lax.axis_index("c")
    s = lax.axis_index("s")
    rows = (rows0, rows1, rows2, rows3)
    sg = (sg0, sg1, sg2, sg3)
    ss = (ss0, ss1, ss2, ss3)

    @pl.loop(0, CHS)
    def _(r):
        for g in range(DH // L):
            rows0[r, pl.ds(g * L, L)] = jnp.zeros((L,), jnp.float32)
            rows1[r, pl.ds(g * L, L)] = jnp.zeros((L,), jnp.float32)

    @pl.loop(0, RPT // (2 * CHS))
    def _(k):
        pltpu.sync_copy(rows0, acc.at[pl.ds(s * RPT + k * 2 * CHS, CHS)])
        pltpu.sync_copy(rows1, acc.at[pl.ds(s * RPT + k * 2 * CHS + CHS, CHS)])
    plsc.subcore_barrier()

    def stage_idx_async(bb, sv, dv):
        pltpu.async_copy(srcoff_hbm.at[c, s, pl.ds(bb * NB, NB)], sv, si)
        pltpu.async_copy(dst_hbm.at[s, pl.ds(bb * NB, NB)], dv, si)

    def wait_idx(sv, dv):
        pltpu.make_async_copy(srcoff_hbm.at[c, s, pl.ds(0, NB)], sv, si).wait()
        pltpu.make_async_copy(dst_hbm.at[s, pl.ds(0, NB)], dv, si).wait()

    def gather_issue(idx_row, p):
        pltpu.async_copy(table_hbm.at[idx_row], rows[p], sg[p])

    def gather_wait(sv, p):
        pltpu.make_async_copy(table_hbm.at[pl.ds(0, CHS)], rows[p], sg[p]).wait()

    def scatter_issue(dx, jj, p):
        pltpu.async_copy(rows[p], acc.at[dx.at[jj]], ss[p], add=True)

    def scatter_wait(dx, p):
        pltpu.make_async_copy(rows[p], acc.at[pl.ds(0, CHS)], ss[p]).wait()

    LA = DEPTH - 1

    def do_block(bb, sx, dx, sy, dy, prefetch, last, first=False):
        if prefetch:
            stage_idx_async(bb + 1, sy, dy)
        for jj in range(NB):
            p = jj % DEPTH
            if jj == NB - LA and prefetch:
                wait_idx(sy, dy)
            gather_wait(sx, p)
            if not (last and jj >= NB - LA):
                q = (jj + LA) % DEPTH
                if not (first and jj == 0):
                    scatter_wait(dx, q)
                nidx = sx.at[jj + LA] if jj < NB - LA else sy.at[jj - (NB - LA)]
                gather_issue(nidx, q)
            scatter_issue(dx, jj, p)
        if last:
            for p in range(DEPTH):
                scatter_wait(dx, p)

    pltpu.sync_copy(srcoff_hbm.at[c, s, pl.ds(0, NB)], srcA)
    pltpu.sync_copy(dst_hbm.at[s, pl.ds(0, NB)], dstA)
    for j0 in range(LA):
        gather_issue(srcA.at[j0], j0)

    do_block(0, srcA, dstA, srcB, dstB, prefetch=True, last=False, first=True)

    @pl.loop(0, NBLK // 2 - 1)
    def _(m):
        do_block(2 * m + 1, srcB, dstB, srcA, dstA, prefetch=True, last=False)
        do_block(2 * m + 2, srcA, dstA, srcB, dstB, prefetch=True, last=False)

    do_block(NBLK - 1, srcB, dstB, srcA, dstA, prefetch=False, last=True)

    plsc.subcore_barrier()
    pltpu.sync_copy(acc.at[pl.ds(s * RPT, RPT)],
                    out_hbm.at[c, pl.ds(s * RPT, RPT)])


def _wvec(degs):
    return lax.rsqrt(jnp.maximum(degs[0, :, 0] + degs[1, :, 0], 1.0))


def _split(h):
    return jnp.stack([h[:, :DH], h[:, DH:]], axis=0)


def _linear_scale_body(x_ref, w_ref, b_ref, degs_ref, out_ref):
    h = jnp.dot(x_ref[...], w_ref[...],
                preferred_element_type=jnp.float32) + b_ref[...]
    wo = _wvec(degs_ref[...])
    out_ref[...] = _split(h * wo[:, None])


def _linear_scale(x, W, b, degp_src):
    return pl.pallas_call(
        _linear_scale_body,
        grid=(GR,),
        in_specs=[
            pl.BlockSpec((BR, D), lambda i: (i, 0)),
            pl.BlockSpec((D, D), lambda i: (0, 0)),
            pl.BlockSpec((1, D), lambda i: (0, 0)),
            pl.BlockSpec((NC, BR, CW), lambda i: (0, i, 0)),
        ],
        out_specs=pl.BlockSpec((NC, BR, DH), lambda i: (0, i, 0)),
        out_shape=jax.ShapeDtypeStruct((NC, NP, DH), jnp.float32),
    )(x, W, b.reshape(1, D), degp_src)


def _combine_linear_body(p_ref, degd_ref, degs_ref, w_ref, b_ref, out_ref):
    p = p_ref[...]
    sup = jnp.concatenate([p[0], p[1]], axis=-1)
    wi = _wvec(degd_ref[...])
    h = jnp.maximum(sup * wi[:, None], 0.0)
    wo = _wvec(degs_ref[...])
    out_ref[...] = _split(
        (jnp.dot(h, w_ref[...], preferred_element_type=jnp.float32)
         + b_ref[...]) * wo[:, None])


def _combine_linear(p, degp_dst, degp_src, W, b):
    return pl.pallas_call(
        _combine_linear_body,
        grid=(GR,),
        in_specs=[
            pl.BlockSpec((NC, BR, DH), lambda i: (0, i, 0)),
            pl.BlockSpec((NC, BR, CW), lambda i: (0, i, 0)),
            pl.BlockSpec((NC, BR, CW), lambda i: (0, i, 0)),
            pl.BlockSpec((D, D), lambda i: (0, 0)),
            pl.BlockSpec((1, D), lambda i: (0, 0)),
        ],
        out_specs=pl.BlockSpec((NC, BR, DH), lambda i: (0, i, 0)),
        out_shape=jax.ShapeDtypeStruct((NC, NP, DH), jnp.float32),
    )(p, degp_dst, degp_src, W, b.reshape(1, D))


def _combine_relu_body(p_ref, degd_ref, out_ref):
    p = p_ref[...]
    sup = jnp.concatenate([p[0], p[1]], axis=-1)
    wi = _wvec(degd_ref[...])
    out_ref[...] = jnp.maximum(sup * wi[:, None], 0.0)


def _combine_relu(p, degp_dst):
    return pl.pallas_call(
        _combine_relu_body,
        grid=(GR,),
        in_specs=[
            pl.BlockSpec((NC, BR, DH), lambda i: (0, i, 0)),
            pl.BlockSpec((NC, BR, CW), lambda i: (0, i, 0)),
        ],
        out_specs=pl.BlockSpec((BR, D), lambda i: (i, 0)),
        out_shape=jax.ShapeDtypeStruct((NP, D), jnp.float32),
    )(p, degp_dst)


def kernel(x, edge_index, W1, b1, W2, b2):
    src = edge_index[0]
    dst = edge_index[1]
    fill = jnp.full((EPAD - E,), NP - 1, jnp.int32)
    srcp = jnp.concatenate([src, fill])
    dstp = jnp.concatenate([dst, fill])
    src_d = srcp.reshape(NW, NCHD, CH)
    dst_d = dstp.reshape(NW, NCHD, CH)
    srcoff = (srcp[None, :] + (jnp.arange(NC, dtype=jnp.int32) * NP)[:, None]
              ).reshape(NC, NS, NCHS, CHS)
    dst_s = dstp.reshape(NS, NCHS, CHS)
    xp = jnp.pad(x, ((0, NP - N), (0, 0)))

    degp_src, degp_dst = _deg_kernel(src_d, dst_d)
    h1s = _linear_scale(xp, W1, b1, degp_src)
    p1 = _spmm_kernel(h1s.reshape(NC * NP, DH), srcoff, dst_s)
    h2s = _combine_linear(p1, degp_dst, degp_src, W2, b2)
    p2 = _spmm_kernel(h2s.reshape(NC * NP, DH), srcoff, dst_s)
    out = _combine_relu(p2, degp_dst)
    return out[:N]
